# baseline (speedup 1.0000x reference)
import jax
import jax.numpy as jnp
from jax.experimental import pallas as pl
from jax.experimental.pallas import tpu as pltpu


def _bcast_kernel(tid_ref, src_ref, out_ref):
    del tid_ref
    out_ref[...] = jnp.broadcast_to(src_ref[...], out_ref.shape)


def kernel(x_query, vis_mark, e_p, task_id):
    del vis_mark
    B = x_query.shape[0]
    nL, _, P, D = e_p.shape
    tid = jnp.asarray(task_id, jnp.int32).reshape((1,))
    b_blk = 32
    grid = (nL, B // b_blk)
    return pl.pallas_call(
        _bcast_kernel,
        grid_spec=pltpu.PrefetchScalarGridSpec(
            num_scalar_prefetch=1,
            grid=grid,
            in_specs=[
                pl.BlockSpec((1, 1, P, D), lambda l, b, tid: (l, tid[0], 0, 0)),
            ],
            out_specs=pl.BlockSpec((1, b_blk, P, D), lambda l, b, tid: (l, b, 0, 0)),
        ),
        out_shape=jax.ShapeDtypeStruct((nL, B, P, D), e_p.dtype),
        compiler_params=pltpu.CompilerParams(
            dimension_semantics=("parallel", "parallel"),
        ),
    )(tid, e_p)


# speedup vs baseline: 1.0784x; 1.0784x over previous
import jax
import jax.numpy as jnp
from jax.experimental import pallas as pl
from jax.experimental.pallas import tpu as pltpu

_R = 16
_NQ = 8


def _dma_kernel(tid_ref, ep_ref, out_ref, sel_buf, big_buf, gsem, osems):
    tid = tid_ref[0]
    gcp = pltpu.make_async_copy(ep_ref.at[:, tid], sel_buf, gsem)
    gcp.start()
    gcp.wait()
    big_buf[...] = jnp.broadcast_to(sel_buf[...][:, None], big_buf.shape)
    for k in range(_NQ):
        pltpu.make_async_copy(
            big_buf, out_ref.at[:, k * _R:(k + 1) * _R], osems.at[k]
        ).start()
    for k in range(_NQ):
        pltpu.make_async_copy(
            big_buf, out_ref.at[:, k * _R:(k + 1) * _R], osems.at[k]
        ).wait()


def kernel(x_query, vis_mark, e_p, task_id):
    del vis_mark
    B = x_query.shape[0]
    nL, _, P, D = e_p.shape
    assert B == _R * _NQ
    tid = jnp.asarray(task_id, jnp.int32).reshape((1,))
    return pl.pallas_call(
        _dma_kernel,
        grid_spec=pltpu.PrefetchScalarGridSpec(
            num_scalar_prefetch=1,
            grid=(1,),
            in_specs=[pl.BlockSpec(memory_space=pl.ANY)],
            out_specs=pl.BlockSpec(memory_space=pl.ANY),
            scratch_shapes=[
                pltpu.VMEM((nL, P, D), jnp.float32),
                pltpu.VMEM((nL, _R, P, D), jnp.float32),
                pltpu.SemaphoreType.DMA,
                pltpu.SemaphoreType.DMA((_NQ,)),
            ],
        ),
        out_shape=jax.ShapeDtypeStruct((nL, B, P, D), e_p.dtype),
    )(tid, e_p)
